# baseline (speedup 1.0000x reference)
import functools

import jax
import jax.numpy as jnp
from jax import lax
from jax.experimental import pallas as pl
from jax.experimental.pallas import tpu as pltpu
from jax.experimental.pallas import tpu_sc as plsc

F = 26
V = 100000
D = 32
B = 16384
VP = 100096
NW2 = F * VP
TC_FULL = 781


def _make_widen():
    info = plsc.get_sparse_core_info()
    NC = info.num_cores
    mesh = plsc.VectorSubcoreMesh(core_axis_name="c", subcore_axis_name="s")

    @functools.partial(
        pl.kernel,
        mesh=mesh,
        out_type=jax.ShapeDtypeStruct((NW2, 128), jnp.float32),
        scratch_types=[
            pltpu.VMEM((2, 4, 8, 128), jnp.float32),
            pltpu.VMEM((2, 128, 128), jnp.float32),
            pltpu.SemaphoreType.DMA,
            pltpu.SemaphoreType.DMA,
            pltpu.SemaphoreType.DMA,
            pltpu.SemaphoreType.DMA,
        ],
        compiler_params=pltpu.CompilerParams(needs_layout_passes=False),
    )
    def widen(tab_t, tail, w_out, tbuf, rbuf, s_in0, s_in1, s_out0, s_out1):
        wid = lax.axis_index("s") * NC + lax.axis_index("c")
        lo_c = wid * 24 + jnp.minimum(wid, 13)
        nj = jnp.where(wid < 13, 25, 24)
        sins = (s_in0, s_in1)
        souts = (s_out0, s_out1)

        def fire_in(par, f, c):
            for a in range(4):
                pltpu.async_copy(
                    tab_t.at[f, pl.ds(a * 8, 8), pl.ds(c * 128, 128)],
                    tbuf.at[par, a],
                    sins[par],
                )

        def drain_in(par):
            for a in range(4):
                pltpu.make_async_copy(
                    tab_t.at[0, pl.ds(0, 8), pl.ds(0, 128)], tbuf.at[0, a], sins[par]
                ).wait()

        def drain_out(par):
            pltpu.make_async_copy(
                w_out.at[pl.ds(0, 128)], rbuf.at[0], souts[par]
            ).wait()

        def body(k, f, c, j):
            fn = jnp.where(f < F - 1, f + 1, 0)
            cn = jnp.minimum(jnp.where(f < F - 1, c, c + 1), TC_FULL - 1)
            for par in range(2):
                @pl.when((k + 1) % 2 == par)
                def _():
                    fire_in(par, fn, cn)

            for par in range(2):
                @pl.when(k % 2 == par)
                def _():
                    drain_in(par)

                    @pl.when(k >= 2)
                    def _():
                        drain_out(par)

                    for a in range(4):
                        for vc in range(8):
                            v16 = lax.iota(jnp.int32, 16) + vc * 16
                            for e in range(8):
                                vals = plsc.load_gather(
                                    tbuf.at[par, a],
                                    [jnp.full((16,), e, jnp.int32), v16],
                                )
                                plsc.store_scatter(
                                    rbuf.at[par],
                                    [v16, jnp.full((16,), a * 8 + e, jnp.int32)],
                                    vals,
                                )
                    pltpu.async_copy(
                        rbuf.at[par],
                        w_out.at[pl.ds(f * VP + c * 128, 128)],
                        souts[par],
                    )

            return k + 1, f, c

        def f_loop(j, carry):
            c = lo_c + j

            def inner(f, carry):
                k, fp, cp = carry
                k2, _, _ = body(k, f, c, j)
                return (k2, f, c)

            return lax.fori_loop(0, F, inner, carry)

        fire_in(0, 0, lo_c)
        k, fp, cp = lax.fori_loop(
            0, nj, f_loop, (jnp.int32(0), jnp.int32(0), jnp.int32(0))
        )

        for par in range(2):
            @pl.when(k % 2 == par)
            def _():
                drain_in(par)

        for d in (1, 2):
            @pl.when(k >= d)
            def _():
                for par in range(2):
                    @pl.when((k - d) % 2 == par)
                    def _():
                        drain_out(par)

        for f in range(F):
            @pl.when(wid == f)
            def _():
                pltpu.sync_copy(
                    tail.at[pl.ds(f * 32, 32)],
                    w_out.at[pl.ds(f * VP + TC_FULL * 128, 32)],
                )

    return widen


def _make_emb_kernel():
    info = plsc.get_sparse_core_info()
    NC, NS = info.num_cores, info.num_subcores
    NW = NC * NS
    BPW = B // NW
    HC = 256

    mesh = plsc.VectorSubcoreMesh(core_axis_name="c", subcore_axis_name="s")

    @functools.partial(
        pl.kernel,
        mesh=mesh,
        out_type=jax.ShapeDtypeStruct((F * B, 128), jnp.float32),
        scratch_types=[
            pltpu.VMEM((BPW, F), jnp.int32),
            pltpu.VMEM((F, BPW), jnp.int32),
            pltpu.VMEM((2, HC, 128), jnp.float32),
            pltpu.SemaphoreType.DMA,
            pltpu.SemaphoreType.DMA,
            pltpu.SemaphoreType.DMA,
            pltpu.SemaphoreType.DMA,
        ],
        compiler_params=pltpu.CompilerParams(
            use_tc_tiling_on_sc=False, needs_layout_passes=False
        ),
    )
    def emb(fv_hbm, tab_hbm, out_hbm, fv_v, idx_v, rows_v, sg0, sg1, sw0, sw1):
        wid = lax.axis_index("s") * NC + lax.axis_index("c")
        base = wid * BPW
        pltpu.sync_copy(fv_hbm.at[pl.ds(base, BPW)], fv_v)

        def tr_body(j, carry):
            rows = lax.iota(jnp.int32, 16) + j * 16
            for i in range(F):
                col = jnp.full((16,), i, jnp.int32)
                v = plsc.load_gather(fv_v, [rows, col]) + (i * VP)
                idx_v[i, pl.ds(pl.multiple_of(j * 16, 16), 16)] = v
            return carry

        lax.fori_loop(0, BPW // 16, tr_body, 0)
        sg = (sg0, sg1)
        sw = (sw0, sw1)
        wb = [None, None]
        step = 0
        for i in range(F):
            for h in range(BPW // HC):
                b = step % 2
                if wb[b] is not None:
                    wb[b].wait()
                pltpu.async_copy(
                    tab_hbm.at[idx_v.at[i, pl.ds(h * HC, HC)]],
                    rows_v.at[b],
                    sg[b],
                ).wait()
                wb[b] = pltpu.async_copy(
                    rows_v.at[b],
                    out_hbm.at[pl.ds(i * B + base + h * HC, HC)],
                    sw[b],
                )
                step += 1
        wb[0].wait()
        wb[1].wait()

    return emb


def kernel(feature_value, tables):
    tab_t = jnp.transpose(tables, (0, 2, 1))
    tail = jnp.pad(
        tables[:, TC_FULL * 128 :, :].reshape(F * 32, D), ((0, 0), (0, 96))
    )
    tab_wide = _make_widen()(tab_t, tail)
    s = _make_emb_kernel()(feature_value, tab_wide)
    return s.reshape(F, B, 128)[:, :, :D]


# speedup vs baseline: 1.6739x; 1.6739x over previous
import functools

import jax
import jax.numpy as jnp
from jax import lax
from jax.experimental import pallas as pl
from jax.experimental.pallas import tpu as pltpu
from jax.experimental.pallas import tpu_sc as plsc


def _make_emb_kernel(F, V, D, B):
    info = plsc.get_sparse_core_info()
    NC, NS = info.num_cores, info.num_subcores
    NW = NC * NS
    assert B % NW == 0
    BPW = B // NW

    mesh = plsc.VectorSubcoreMesh(core_axis_name="c", subcore_axis_name="s")

    @functools.partial(
        pl.kernel,
        mesh=mesh,
        out_type=jax.ShapeDtypeStruct((F * B, 128), jnp.float32),
        scratch_types=[
            pltpu.VMEM((BPW, F), jnp.int32),
            pltpu.VMEM((F, BPW), jnp.int32),
            pltpu.VMEM((2, BPW, D), jnp.float32),
            pltpu.SemaphoreType.DMA,
            pltpu.SemaphoreType.DMA,
            pltpu.SemaphoreType.DMA,
            pltpu.SemaphoreType.DMA,
        ],
        compiler_params=pltpu.CompilerParams(
            use_tc_tiling_on_sc=False, needs_layout_passes=False
        ),
    )
    def emb(fv_hbm, tab_hbm, out_hbm, fv_v, idx_v, rows_v, sg0, sg1, sw0, sw1):
        wid = lax.axis_index("s") * NC + lax.axis_index("c")
        base = wid * BPW
        pltpu.sync_copy(fv_hbm.at[pl.ds(base, BPW)], fv_v)

        def tr_body(j, carry):
            rows = lax.iota(jnp.int32, 16) + j * 16
            for i in range(F):
                col = jnp.full((16,), i, jnp.int32)
                v = plsc.load_gather(fv_v, [rows, col])
                idx_v[i, pl.ds(pl.multiple_of(j * 16, 16), 16)] = v
            return carry

        lax.fori_loop(0, BPW // 16, tr_body, 0)
        sg = (sg0, sg1)
        sw = (sw0, sw1)
        wb = [None, None]
        for i in range(F):
            b = i % 2
            if wb[b] is not None:
                wb[b].wait()
            pltpu.async_copy(tab_hbm.at[i].at[idx_v.at[i]], rows_v.at[b], sg[b]).wait()
            wb[b] = pltpu.async_copy(
                rows_v.at[b],
                out_hbm.at[pl.ds(i * B + base, BPW), pl.ds(0, D)],
                sw[b],
            )
        wb[0].wait()
        wb[1].wait()

    return emb


def kernel(feature_value, tables):
    F, V, D = tables.shape
    B = feature_value.shape[0]
    emb = _make_emb_kernel(F, V, D, B)
    s = emb(feature_value, tables)
    return s.reshape(F, B, 128)[:, :, :D]
